# Initial kernel scaffold; baseline (speedup 1.0000x reference)
import jax
import jax.numpy as jnp
from jax.experimental import pallas as pl


def kernel(proposals, gt_boxes, gt_labels):
    raise NotImplementedError("write your pallas kernel here")


# speedup vs baseline: 2.0778x; 2.0778x over previous
import functools

import jax
import jax.numpy as jnp
from jax import lax
from jax.experimental import pallas as pl
from jax.experimental.pallas import tpu as pltpu
from jax.experimental.pallas import tpu_sc as plsc

L = 16
NW = 32
FG_BG_THRESH = 0.5


def _make_sc_call(npad, g):
    chunk = npad // NW
    nblk = chunk // L
    mesh = plsc.VectorSubcoreMesh(core_axis_name="c", subcore_axis_name="s")

    @functools.partial(
        pl.kernel,
        mesh=mesh,
        compiler_params=pltpu.CompilerParams(needs_layout_passes=False),
        out_type=[
            jax.ShapeDtypeStruct((npad,), jnp.int32),
            jax.ShapeDtypeStruct((npad,), jnp.float32),
            jax.ShapeDtypeStruct((npad,), jnp.int32),
        ],
        scratch_types=[
            pltpu.VMEM((chunk * 4,), jnp.float32),
            pltpu.VMEM((g * 4,), jnp.float32),
            pltpu.VMEM((g,), jnp.int32),
            pltpu.VMEM((chunk,), jnp.int32),
            pltpu.VMEM((chunk,), jnp.float32),
            pltpu.VMEM((chunk,), jnp.int32),
        ],
    )
    def sc_call(props_hbm, gtb_hbm, gtl_hbm,
                lab_hbm, mv_hbm, idx_hbm,
                props_v, gtb_v, gtl_v, lab_v, mv_v, idx_v):
        cid = lax.axis_index("c")
        sid = lax.axis_index("s")
        wid = sid * 2 + cid
        base = wid * chunk

        pltpu.sync_copy(gtb_hbm, gtb_v)
        pltpu.sync_copy(gtl_hbm, gtl_v)
        pltpu.sync_copy(props_hbm.at[pl.ds(base * 4, chunk * 4)], props_v)

        def block(j, _):
            rows4 = (lax.iota(jnp.int32, L) + j * L) * 4
            px0 = plsc.load_gather(props_v, [rows4])
            py0 = plsc.load_gather(props_v, [rows4 + 1])
            px1 = plsc.load_gather(props_v, [rows4 + 2])
            py1 = plsc.load_gather(props_v, [rows4 + 3])
            parea = (px1 - px0) * (py1 - py0)

            def giter(gi, carry):
                best, bidx = carry
                rows_g = jnp.full((L,), gi * 4, jnp.int32)
                gx0 = plsc.load_gather(gtb_v, [rows_g])
                gy0 = plsc.load_gather(gtb_v, [rows_g + 1])
                gx1 = plsc.load_gather(gtb_v, [rows_g + 2])
                gy1 = plsc.load_gather(gtb_v, [rows_g + 3])
                garea = (gx1 - gx0) * (gy1 - gy0)
                w = jnp.maximum(
                    jnp.minimum(px1, gx1) - jnp.maximum(px0, gx0), 0.0)
                h = jnp.maximum(
                    jnp.minimum(py1, gy1) - jnp.maximum(py0, gy0), 0.0)
                inter = w * h
                union = (parea + garea) - inter
                iou = inter / union
                upd = iou > best
                best = jnp.where(upd, iou, best)
                bidx = jnp.where(upd, jnp.full((L,), gi, jnp.int32), bidx)
                return best, bidx

            best, bidx = lax.fori_loop(
                0, g, giter,
                (jnp.full((L,), -1.0, jnp.float32), jnp.zeros((L,), jnp.int32)))

            below = best < FG_BG_THRESH
            zi = jnp.zeros((L,), jnp.int32)
            cidx = jnp.where(below, zi, bidx)
            labs = plsc.load_gather(gtl_v, [cidx])
            labs = jnp.where(below, zi, labs)
            mv_v[pl.ds(j * L, L)] = best
            idx_v[pl.ds(j * L, L)] = cidx
            lab_v[pl.ds(j * L, L)] = labs
            return 0

        lax.fori_loop(0, nblk, block, 0)

        pltpu.sync_copy(lab_v, lab_hbm.at[pl.ds(base, chunk)])
        pltpu.sync_copy(mv_v, mv_hbm.at[pl.ds(base, chunk)])
        pltpu.sync_copy(idx_v, idx_hbm.at[pl.ds(base, chunk)])

    return sc_call


def kernel(proposals, gt_boxes, gt_labels):
    n = proposals.shape[0]
    g = gt_boxes.shape[0]
    npad = -(-n // (NW * L)) * (NW * L)
    props = jnp.concatenate(
        [proposals, jnp.zeros((npad - n, 4), jnp.float32)], axis=0)
    lab, mv, idx = _make_sc_call(npad, g)(
        props.reshape(-1), gt_boxes.reshape(-1), gt_labels)
    return lab[:n], mv[:n], idx[:n]


# speedup vs baseline: 2.1399x; 1.0299x over previous
import functools

import jax
import jax.numpy as jnp
from jax import lax
from jax.experimental import pallas as pl
from jax.experimental.pallas import tpu as pltpu
from jax.experimental.pallas import tpu_sc as plsc

L = 16
NW = 32
PB = 2
FG_BG_THRESH = 0.5


def _make_sc_call(npad, g):
    chunk = npad // NW
    nblk = chunk // L
    mesh = plsc.VectorSubcoreMesh(core_axis_name="c", subcore_axis_name="s")

    @functools.partial(
        pl.kernel,
        mesh=mesh,
        compiler_params=pltpu.CompilerParams(needs_layout_passes=False),
        out_type=[
            jax.ShapeDtypeStruct((npad,), jnp.int32),
            jax.ShapeDtypeStruct((npad,), jnp.float32),
            jax.ShapeDtypeStruct((npad,), jnp.int32),
        ],
        scratch_types=[
            pltpu.VMEM((chunk * 4,), jnp.float32),
            pltpu.VMEM((g * 4,), jnp.float32),
            pltpu.VMEM((g,), jnp.int32),
            pltpu.VMEM((chunk,), jnp.int32),
            pltpu.VMEM((chunk,), jnp.float32),
            pltpu.VMEM((chunk,), jnp.int32),
        ],
    )
    def sc_call(props_hbm, gtb_hbm, gtl_hbm,
                lab_hbm, mv_hbm, idx_hbm,
                props_v, gtb_v, gtl_v, lab_v, mv_v, idx_v):
        cid = lax.axis_index("c")
        sid = lax.axis_index("s")
        wid = sid * 2 + cid
        base = wid * chunk

        pltpu.sync_copy(gtb_hbm, gtb_v)
        pltpu.sync_copy(gtl_hbm, gtl_v)
        pltpu.sync_copy(props_hbm.at[pl.ds(base * 4, chunk * 4)], props_v)

        def block(jb, _):
            px0 = [None] * PB
            py0 = [None] * PB
            px1 = [None] * PB
            py1 = [None] * PB
            parea = [None] * PB
            for a in range(PB):
                rows4 = (lax.iota(jnp.int32, L) + (jb * PB + a) * L) * 4
                px0[a] = plsc.load_gather(props_v, [rows4])
                py0[a] = plsc.load_gather(props_v, [rows4 + 1])
                px1[a] = plsc.load_gather(props_v, [rows4 + 2])
                py1[a] = plsc.load_gather(props_v, [rows4 + 3])
                parea[a] = (px1[a] - px0[a]) * (py1[a] - py0[a])

            def giter(gi, carry):
                best = list(carry[:PB])
                bidx = list(carry[PB:])
                rows_g = jnp.full((L,), gi * 4, jnp.int32)
                gx0 = plsc.load_gather(gtb_v, [rows_g])
                gy0 = plsc.load_gather(gtb_v, [rows_g + 1])
                gx1 = plsc.load_gather(gtb_v, [rows_g + 2])
                gy1 = plsc.load_gather(gtb_v, [rows_g + 3])
                garea = (gx1 - gx0) * (gy1 - gy0)
                gidx = jnp.full((L,), gi, jnp.int32)
                for a in range(PB):
                    w = jnp.maximum(
                        jnp.minimum(px1[a], gx1) - jnp.maximum(px0[a], gx0),
                        0.0)
                    h = jnp.maximum(
                        jnp.minimum(py1[a], gy1) - jnp.maximum(py0[a], gy0),
                        0.0)
                    inter = w * h
                    union = (parea[a] + garea) - inter
                    iou = inter / union
                    upd = iou > best[a]
                    best[a] = jnp.where(upd, iou, best[a])
                    bidx[a] = jnp.where(upd, gidx, bidx[a])
                return tuple(best) + tuple(bidx)

            init = ((jnp.full((L,), -1.0, jnp.float32),) * PB
                    + (jnp.zeros((L,), jnp.int32),) * PB)
            res = lax.fori_loop(0, g, giter, init)

            zi = jnp.zeros((L,), jnp.int32)
            for a in range(PB):
                best, bidx = res[a], res[PB + a]
                below = best < FG_BG_THRESH
                cidx = jnp.where(below, zi, bidx)
                labs = plsc.load_gather(gtl_v, [cidx])
                labs = jnp.where(below, zi, labs)
                off = (jb * PB + a) * L
                mv_v[pl.ds(off, L)] = best
                idx_v[pl.ds(off, L)] = cidx
                lab_v[pl.ds(off, L)] = labs
            return 0

        lax.fori_loop(0, nblk // PB, block, 0)

        pltpu.sync_copy(lab_v, lab_hbm.at[pl.ds(base, chunk)])
        pltpu.sync_copy(mv_v, mv_hbm.at[pl.ds(base, chunk)])
        pltpu.sync_copy(idx_v, idx_hbm.at[pl.ds(base, chunk)])

    return sc_call


def kernel(proposals, gt_boxes, gt_labels):
    n = proposals.shape[0]
    g = gt_boxes.shape[0]
    npad = -(-n // (NW * L)) * (NW * L)
    props = jnp.concatenate(
        [proposals, jnp.zeros((npad - n, 4), jnp.float32)], axis=0)
    lab, mv, idx = _make_sc_call(npad, g)(
        props.reshape(-1), gt_boxes.reshape(-1), gt_labels)
    return lab[:n], mv[:n], idx[:n]


# speedup vs baseline: 2.5674x; 1.1998x over previous
import functools

import jax
import jax.numpy as jnp
from jax import lax
from jax.experimental import pallas as pl
from jax.experimental.pallas import tpu as pltpu
from jax.experimental.pallas import tpu_sc as plsc

L = 16
NW = 32
FG_BG_THRESH = 0.5


def _make_sc_call(n, g):
    chunk = 640
    last = n - (NW - 1) * chunk
    assert last > 0 and last % L == 0 and chunk % L == 0
    nblk = chunk // L
    nblk_last = last // L
    mesh = plsc.VectorSubcoreMesh(core_axis_name="c", subcore_axis_name="s")

    steps = []
    s = g // 2
    while s >= 1:
        steps.append(s)
        s //= 2

    @functools.partial(
        pl.kernel,
        mesh=mesh,
        compiler_params=pltpu.CompilerParams(needs_layout_passes=False),
        out_type=[
            jax.ShapeDtypeStruct((n,), jnp.int32),
            jax.ShapeDtypeStruct((n,), jnp.float32),
            jax.ShapeDtypeStruct((n,), jnp.int32),
        ],
        scratch_types=[
            pltpu.VMEM((chunk, 4), jnp.float32),
            pltpu.VMEM((g,), jnp.float32),
            pltpu.VMEM((g,), jnp.float32),
            pltpu.VMEM((g,), jnp.float32),
            pltpu.VMEM((g,), jnp.float32),
            pltpu.VMEM((g,), jnp.float32),
            pltpu.VMEM((g,), jnp.int32),
            pltpu.VMEM((g,), jnp.int32),
            pltpu.VMEM((chunk,), jnp.int32),
            pltpu.VMEM((chunk,), jnp.float32),
            pltpu.VMEM((chunk,), jnp.int32),
        ],
    )
    def sc_call(props_hbm, gtf_hbm, gti_hbm,
                lab_hbm, mv_hbm, idx_hbm,
                props_v, gx0_v, gy0_v, gx1_v, gy1_v, ga_v, orig_v, gtl_v,
                lab_v, mv_v, idx_v):
        cid = lax.axis_index("c")
        sid = lax.axis_index("s")
        wid = sid * 2 + cid
        base = wid * chunk
        is_last = wid == NW - 1

        pltpu.sync_copy(gtf_hbm.at[pl.ds(0, g)], gx0_v)
        pltpu.sync_copy(gtf_hbm.at[pl.ds(g, g)], gy0_v)
        pltpu.sync_copy(gtf_hbm.at[pl.ds(2 * g, g)], gx1_v)
        pltpu.sync_copy(gtf_hbm.at[pl.ds(3 * g, g)], gy1_v)
        pltpu.sync_copy(gti_hbm.at[pl.ds(0, g)], orig_v)
        pltpu.sync_copy(gti_hbm.at[pl.ds(g, g)], gtl_v)

        @pl.when(jnp.logical_not(is_last))
        def _():
            pltpu.sync_copy(props_hbm.at[pl.ds(base, chunk)],
                            props_v.at[pl.ds(0, chunk)])

        @pl.when(is_last)
        def _():
            pltpu.sync_copy(props_hbm.at[pl.ds(base, last)],
                            props_v.at[pl.ds(0, last)])

        mw = jnp.zeros((L,), jnp.float32)
        for i in range(g // L):
            x0v = gx0_v[pl.ds(i * L, L)]
            x1v = gx1_v[pl.ds(i * L, L)]
            y0v = gy0_v[pl.ds(i * L, L)]
            y1v = gy1_v[pl.ds(i * L, L)]
            wv = x1v - x0v
            ga_v[pl.ds(i * L, L)] = wv * (y1v - y0v)
            mw = jnp.maximum(mw, wv)
        maxw = jnp.max(mw)

        zi = jnp.zeros((L,), jnp.int32)
        zf = jnp.zeros((L,), jnp.float32)
        c0 = zi

        def block(j, _):
            rows = lax.iota(jnp.int32, L) + j * L
            px0 = plsc.load_gather(props_v, [rows, c0])
            py0 = plsc.load_gather(props_v, [rows, c0 + 1])
            px1 = plsc.load_gather(props_v, [rows, c0 + 2])
            py1 = plsc.load_gather(props_v, [rows, c0 + 3])
            parea = (px1 - px0) * (py1 - py0)

            tlo = px0 - maxw
            lo = zi
            hi = zi
            for s in steps:
                vlo = plsc.load_gather(gx0_v, [lo + (s - 1)])
                lo = jnp.where(vlo < tlo, lo + s, lo)
                vhi = plsc.load_gather(gx0_v, [hi + (s - 1)])
                hi = jnp.where(vhi <= px1, hi + s, hi)
            vlo = plsc.load_gather(gx0_v, [lo])
            lo = jnp.where(vlo < tlo, lo + 1, lo)
            vhi = plsc.load_gather(gx0_v, [hi])
            hi = jnp.where(vhi <= px1, hi + 1, hi)

            trip = jnp.max(hi - lo)

            def citer(k, carry):
                best, bidx = carry
                idx = lo + k
                m = idx < hi
                safe = jnp.where(m, idx, zi)
                gx0 = plsc.load_gather(gx0_v, [safe])
                gy0 = plsc.load_gather(gy0_v, [safe])
                gx1 = plsc.load_gather(gx1_v, [safe])
                gy1 = plsc.load_gather(gy1_v, [safe])
                ga = plsc.load_gather(ga_v, [safe])
                w = jnp.maximum(
                    jnp.minimum(px1, gx1) - jnp.maximum(px0, gx0), 0.0)
                h = jnp.maximum(
                    jnp.minimum(py1, gy1) - jnp.maximum(py0, gy0), 0.0)
                inter = w * h
                union = (parea + ga) - inter
                iou = inter / union
                upd = m & (iou > best)
                best = jnp.where(upd, iou, best)
                bidx = jnp.where(upd, safe, bidx)
                return best, bidx

            best, bidx = lax.fori_loop(0, trip, citer, (zf, zi))

            below = best < FG_BG_THRESH
            labs = plsc.load_gather(gtl_v, [bidx])
            orig = plsc.load_gather(orig_v, [bidx])
            cidx = jnp.where(below, zi, orig)
            labs = jnp.where(below, zi, labs)
            off = j * L
            mv_v[pl.ds(off, L)] = best
            idx_v[pl.ds(off, L)] = cidx
            lab_v[pl.ds(off, L)] = labs
            return 0

        lax.fori_loop(0, jnp.where(is_last, nblk_last, nblk), block, 0)

        @pl.when(jnp.logical_not(is_last))
        def _():
            pltpu.sync_copy(lab_v.at[pl.ds(0, chunk)],
                            lab_hbm.at[pl.ds(base, chunk)])
            pltpu.sync_copy(mv_v.at[pl.ds(0, chunk)],
                            mv_hbm.at[pl.ds(base, chunk)])
            pltpu.sync_copy(idx_v.at[pl.ds(0, chunk)],
                            idx_hbm.at[pl.ds(base, chunk)])

        @pl.when(is_last)
        def _():
            pltpu.sync_copy(lab_v.at[pl.ds(0, last)],
                            lab_hbm.at[pl.ds(base, last)])
            pltpu.sync_copy(mv_v.at[pl.ds(0, last)],
                            mv_hbm.at[pl.ds(base, last)])
            pltpu.sync_copy(idx_v.at[pl.ds(0, last)],
                            idx_hbm.at[pl.ds(base, last)])

    return sc_call


def kernel(proposals, gt_boxes, gt_labels):
    n = proposals.shape[0]
    g = gt_boxes.shape[0]
    order = jnp.argsort(gt_boxes[:, 0]).astype(jnp.int32)
    gts = gt_boxes[order]
    gtf = jnp.concatenate([gts[:, 0], gts[:, 1], gts[:, 2], gts[:, 3]])
    gti = jnp.concatenate([order, jnp.take(gt_labels, order)])
    lab, mv, idx = _make_sc_call(n, g)(proposals, gtf, gti)
    return lab, mv, idx


# speedup vs baseline: 3.0333x; 1.1815x over previous
import functools

import jax
import jax.numpy as jnp
from jax import lax
from jax.experimental import pallas as pl
from jax.experimental.pallas import tpu as pltpu
from jax.experimental.pallas import tpu_sc as plsc

L = 16
NW = 32
FG_BG_THRESH = 0.5


def _make_sc_call(n, g):
    chunk = 640
    last = n - (NW - 1) * chunk
    assert last > 0 and last % L == 0 and chunk % L == 0
    nblk = chunk // L
    nblk_last = last // L
    mesh = plsc.VectorSubcoreMesh(core_axis_name="c", subcore_axis_name="s")

    steps = []
    s = g // 2
    while s >= 1:
        steps.append(s)
        s //= 2

    @functools.partial(
        pl.kernel,
        mesh=mesh,
        compiler_params=pltpu.CompilerParams(needs_layout_passes=False),
        out_type=[
            jax.ShapeDtypeStruct((n,), jnp.int32),
            jax.ShapeDtypeStruct((n,), jnp.float32),
            jax.ShapeDtypeStruct((n,), jnp.int32),
        ],
        scratch_types=[
            pltpu.VMEM((chunk,), jnp.float32),
            pltpu.VMEM((chunk,), jnp.float32),
            pltpu.VMEM((chunk,), jnp.float32),
            pltpu.VMEM((chunk,), jnp.float32),
            pltpu.VMEM((g,), jnp.float32),
            pltpu.VMEM((g,), jnp.float32),
            pltpu.VMEM((g,), jnp.float32),
            pltpu.VMEM((g,), jnp.float32),
            pltpu.VMEM((g,), jnp.float32),
            pltpu.VMEM((g,), jnp.int32),
            pltpu.VMEM((g,), jnp.int32),
            pltpu.VMEM((chunk,), jnp.int32),
            pltpu.VMEM((chunk,), jnp.float32),
            pltpu.VMEM((chunk,), jnp.int32),
        ],
    )
    def sc_call(px0_hbm, py0_hbm, px1_hbm, py1_hbm, gtf_hbm, gti_hbm,
                lab_hbm, mv_hbm, idx_hbm,
                px0_v, py0_v, px1_v, py1_v,
                gx0_v, gy0_v, gx1_v, gy1_v, ga_v, orig_v, gtl_v,
                lab_v, mv_v, idx_v):
        cid = lax.axis_index("c")
        sid = lax.axis_index("s")
        wid = sid * 2 + cid
        base = wid * chunk
        is_last = wid == NW - 1

        pltpu.sync_copy(gtf_hbm.at[pl.ds(0, g)], gx0_v)
        pltpu.sync_copy(gtf_hbm.at[pl.ds(g, g)], gy0_v)
        pltpu.sync_copy(gtf_hbm.at[pl.ds(2 * g, g)], gx1_v)
        pltpu.sync_copy(gtf_hbm.at[pl.ds(3 * g, g)], gy1_v)
        pltpu.sync_copy(gti_hbm.at[pl.ds(0, g)], orig_v)
        pltpu.sync_copy(gti_hbm.at[pl.ds(g, g)], gtl_v)

        @pl.when(jnp.logical_not(is_last))
        def _():
            for hbm, v in ((px0_hbm, px0_v), (py0_hbm, py0_v),
                           (px1_hbm, px1_v), (py1_hbm, py1_v)):
                pltpu.sync_copy(hbm.at[pl.ds(base, chunk)],
                                v.at[pl.ds(0, chunk)])

        @pl.when(is_last)
        def _():
            for hbm, v in ((px0_hbm, px0_v), (py0_hbm, py0_v),
                           (px1_hbm, px1_v), (py1_hbm, py1_v)):
                pltpu.sync_copy(hbm.at[pl.ds(base, last)],
                                v.at[pl.ds(0, last)])

        mw = jnp.zeros((L,), jnp.float32)
        for i in range(g // L):
            x0v = gx0_v[pl.ds(i * L, L)]
            x1v = gx1_v[pl.ds(i * L, L)]
            y0v = gy0_v[pl.ds(i * L, L)]
            y1v = gy1_v[pl.ds(i * L, L)]
            wv = x1v - x0v
            ga_v[pl.ds(i * L, L)] = wv * (y1v - y0v)
            mw = jnp.maximum(mw, wv)
        maxw = jnp.max(mw)

        zi = jnp.zeros((L,), jnp.int32)
        zf = jnp.zeros((L,), jnp.float32)
        c0 = zi

        def block(j, _):
            off = j * L
            px0 = px0_v[pl.ds(off, L)]
            py0 = py0_v[pl.ds(off, L)]
            px1 = px1_v[pl.ds(off, L)]
            py1 = py1_v[pl.ds(off, L)]
            parea = (px1 - px0) * (py1 - py0)

            tlo = px0 - maxw
            lo = zi
            hi = zi
            for s in steps:
                vlo = plsc.load_gather(gx0_v, [lo + (s - 1)])
                lo = jnp.where(vlo < tlo, lo + s, lo)
                vhi = plsc.load_gather(gx0_v, [hi + (s - 1)])
                hi = jnp.where(vhi <= px1, hi + s, hi)
            vlo = plsc.load_gather(gx0_v, [lo])
            lo = jnp.where(vlo < tlo, lo + 1, lo)
            vhi = plsc.load_gather(gx0_v, [hi])
            hi = jnp.where(vhi <= px1, hi + 1, hi)

            trip = jnp.max(hi - lo)

            def citer(k, carry):
                best, bidx = carry
                idx = lo + k
                m = idx < hi
                safe = jnp.where(m, idx, zi)
                gx0 = plsc.load_gather(gx0_v, [safe])
                gy0 = plsc.load_gather(gy0_v, [safe])
                gx1 = plsc.load_gather(gx1_v, [safe])
                gy1 = plsc.load_gather(gy1_v, [safe])
                ga = plsc.load_gather(ga_v, [safe])
                w = jnp.maximum(
                    jnp.minimum(px1, gx1) - jnp.maximum(px0, gx0), 0.0)
                h = jnp.maximum(
                    jnp.minimum(py1, gy1) - jnp.maximum(py0, gy0), 0.0)
                inter = w * h
                union = (parea + ga) - inter
                iou = inter / union
                upd = m & (iou > best)
                best = jnp.where(upd, iou, best)
                bidx = jnp.where(upd, safe, bidx)
                return best, bidx

            best, bidx = lax.fori_loop(0, trip, citer, (zf, zi))

            below = best < FG_BG_THRESH
            labs = plsc.load_gather(gtl_v, [bidx])
            orig = plsc.load_gather(orig_v, [bidx])
            cidx = jnp.where(below, zi, orig)
            labs = jnp.where(below, zi, labs)
            mv_v[pl.ds(off, L)] = best
            idx_v[pl.ds(off, L)] = cidx
            lab_v[pl.ds(off, L)] = labs
            return 0

        lax.fori_loop(0, jnp.where(is_last, nblk_last, nblk), block, 0)

        @pl.when(jnp.logical_not(is_last))
        def _():
            pltpu.sync_copy(lab_v.at[pl.ds(0, chunk)],
                            lab_hbm.at[pl.ds(base, chunk)])
            pltpu.sync_copy(mv_v.at[pl.ds(0, chunk)],
                            mv_hbm.at[pl.ds(base, chunk)])
            pltpu.sync_copy(idx_v.at[pl.ds(0, chunk)],
                            idx_hbm.at[pl.ds(base, chunk)])

        @pl.when(is_last)
        def _():
            pltpu.sync_copy(lab_v.at[pl.ds(0, last)],
                            lab_hbm.at[pl.ds(base, last)])
            pltpu.sync_copy(mv_v.at[pl.ds(0, last)],
                            mv_hbm.at[pl.ds(base, last)])
            pltpu.sync_copy(idx_v.at[pl.ds(0, last)],
                            idx_hbm.at[pl.ds(base, last)])

    return sc_call


def kernel(proposals, gt_boxes, gt_labels):
    n = proposals.shape[0]
    g = gt_boxes.shape[0]
    order = jnp.argsort(gt_boxes[:, 0]).astype(jnp.int32)
    gts = gt_boxes[order]
    gtf = jnp.concatenate([gts[:, 0], gts[:, 1], gts[:, 2], gts[:, 3]])
    gti = jnp.concatenate([order, jnp.take(gt_labels, order)])
    lab, mv, idx = _make_sc_call(n, g)(
        proposals[:, 0], proposals[:, 1], proposals[:, 2], proposals[:, 3],
        gtf, gti)
    return lab, mv, idx


# speedup vs baseline: 3.0877x; 1.0179x over previous
import functools

import jax
import jax.numpy as jnp
from jax import lax
from jax.experimental import pallas as pl
from jax.experimental.pallas import tpu as pltpu
from jax.experimental.pallas import tpu_sc as plsc

L = 16
NW = 32
FG_BG_THRESH = 0.5


def _make_sc_call(n, g):
    chunk = 640
    last = n - (NW - 1) * chunk
    assert last > 0 and last % L == 0 and chunk % L == 0
    nblk = chunk // L
    nblk_last = last // L
    mesh = plsc.VectorSubcoreMesh(core_axis_name="c", subcore_axis_name="s")

    steps = []
    s = g // 2
    while s >= 1:
        steps.append(s)
        s //= 2

    @functools.partial(
        pl.kernel,
        mesh=mesh,
        compiler_params=pltpu.CompilerParams(needs_layout_passes=False),
        out_type=[
            jax.ShapeDtypeStruct((n,), jnp.int32),
            jax.ShapeDtypeStruct((n,), jnp.float32),
            jax.ShapeDtypeStruct((n,), jnp.int32),
        ],
        scratch_types=[
            pltpu.VMEM((chunk,), jnp.float32),
            pltpu.VMEM((chunk,), jnp.float32),
            pltpu.VMEM((chunk,), jnp.float32),
            pltpu.VMEM((chunk,), jnp.float32),
            pltpu.VMEM((g + L,), jnp.float32),
            pltpu.VMEM((g + L,), jnp.float32),
            pltpu.VMEM((g + L,), jnp.float32),
            pltpu.VMEM((g + L,), jnp.float32),
            pltpu.VMEM((g,), jnp.int32),
            pltpu.VMEM((g,), jnp.int32),
            pltpu.VMEM((chunk,), jnp.int32),
            pltpu.VMEM((chunk,), jnp.float32),
            pltpu.VMEM((chunk,), jnp.int32),
        ],
    )
    def sc_call(px0_hbm, py0_hbm, px1_hbm, py1_hbm, gtf_hbm, gti_hbm,
                lab_hbm, mv_hbm, idx_hbm,
                px0_v, py0_v, px1_v, py1_v,
                gx0_v, gy0_v, gx1_v, gy1_v, orig_v, gtl_v,
                lab_v, mv_v, idx_v):
        cid = lax.axis_index("c")
        sid = lax.axis_index("s")
        wid = sid * 2 + cid
        base = wid * chunk
        is_last = wid == NW - 1

        pltpu.sync_copy(gtf_hbm.at[pl.ds(0, g)], gx0_v.at[pl.ds(0, g)])
        pltpu.sync_copy(gtf_hbm.at[pl.ds(g, g)], gy0_v.at[pl.ds(0, g)])
        pltpu.sync_copy(gtf_hbm.at[pl.ds(2 * g, g)], gx1_v.at[pl.ds(0, g)])
        pltpu.sync_copy(gtf_hbm.at[pl.ds(3 * g, g)], gy1_v.at[pl.ds(0, g)])
        pltpu.sync_copy(gti_hbm.at[pl.ds(0, g)], orig_v)
        pltpu.sync_copy(gti_hbm.at[pl.ds(g, g)], gtl_v)

        @pl.when(jnp.logical_not(is_last))
        def _():
            for hbm, v in ((px0_hbm, px0_v), (py0_hbm, py0_v),
                           (px1_hbm, px1_v), (py1_hbm, py1_v)):
                pltpu.sync_copy(hbm.at[pl.ds(base, chunk)],
                                v.at[pl.ds(0, chunk)])

        @pl.when(is_last)
        def _():
            for hbm, v in ((px0_hbm, px0_v), (py0_hbm, py0_v),
                           (px1_hbm, px1_v), (py1_hbm, py1_v)):
                pltpu.sync_copy(hbm.at[pl.ds(base, last)],
                                v.at[pl.ds(0, last)])

        big = jnp.full((L,), 1.0e30, jnp.float32)
        gx0_v[pl.ds(g, L)] = big
        gy0_v[pl.ds(g, L)] = big
        gx1_v[pl.ds(g, L)] = big
        gy1_v[pl.ds(g, L)] = big

        mw = jnp.zeros((L,), jnp.float32)
        for i in range(g // L):
            x0v = gx0_v[pl.ds(i * L, L)]
            x1v = gx1_v[pl.ds(i * L, L)]
            mw = jnp.maximum(mw, x1v - x0v)
        maxw = jnp.max(mw)

        zi = jnp.zeros((L,), jnp.int32)
        zf = jnp.zeros((L,), jnp.float32)
        c0 = zi

        def block(j, _):
            off = j * L
            px0 = px0_v[pl.ds(off, L)]
            py0 = py0_v[pl.ds(off, L)]
            px1 = px1_v[pl.ds(off, L)]
            py1 = py1_v[pl.ds(off, L)]
            parea = (px1 - px0) * (py1 - py0)

            tlo = px0 - maxw
            lo = zi
            hi = zi
            for s in steps:
                vlo = plsc.load_gather(gx0_v, [lo + (s - 1)])
                lo = jnp.where(vlo < tlo, lo + s, lo)
                vhi = plsc.load_gather(gx0_v, [hi + (s - 1)])
                hi = jnp.where(vhi <= px1, hi + s, hi)
            vlo = plsc.load_gather(gx0_v, [lo])
            lo = jnp.where(vlo < tlo, lo + 1, lo)
            vhi = plsc.load_gather(gx0_v, [hi])
            hi = jnp.where(vhi <= px1, hi + 1, hi)

            trip = jnp.max(hi - lo)
            send = jnp.full((L,), g, jnp.int32)

            def citer(k, carry):
                best, bidx = carry
                safe = jnp.minimum(lo + k, send)
                gx0 = plsc.load_gather(gx0_v, [safe])
                gy0 = plsc.load_gather(gy0_v, [safe])
                gx1 = plsc.load_gather(gx1_v, [safe])
                gy1 = plsc.load_gather(gy1_v, [safe])
                ga = (gx1 - gx0) * (gy1 - gy0)
                w = jnp.maximum(
                    jnp.minimum(px1, gx1) - jnp.maximum(px0, gx0), 0.0)
                h = jnp.maximum(
                    jnp.minimum(py1, gy1) - jnp.maximum(py0, gy0), 0.0)
                inter = w * h
                union = (parea + ga) - inter
                iou = inter / union
                upd = iou > best
                best = jnp.where(upd, iou, best)
                bidx = jnp.where(upd, safe, bidx)
                return best, bidx

            best, bidx = plsc.parallel_loop(
                0, trip, 1, unroll=2, carry=(zf, zi))(citer)

            below = best < FG_BG_THRESH
            labs = plsc.load_gather(gtl_v, [bidx])
            orig = plsc.load_gather(orig_v, [bidx])
            cidx = jnp.where(below, zi, orig)
            labs = jnp.where(below, zi, labs)
            mv_v[pl.ds(off, L)] = best
            idx_v[pl.ds(off, L)] = cidx
            lab_v[pl.ds(off, L)] = labs
            return 0

        lax.fori_loop(0, jnp.where(is_last, nblk_last, nblk), block, 0)

        @pl.when(jnp.logical_not(is_last))
        def _():
            pltpu.sync_copy(lab_v.at[pl.ds(0, chunk)],
                            lab_hbm.at[pl.ds(base, chunk)])
            pltpu.sync_copy(mv_v.at[pl.ds(0, chunk)],
                            mv_hbm.at[pl.ds(base, chunk)])
            pltpu.sync_copy(idx_v.at[pl.ds(0, chunk)],
                            idx_hbm.at[pl.ds(base, chunk)])

        @pl.when(is_last)
        def _():
            pltpu.sync_copy(lab_v.at[pl.ds(0, last)],
                            lab_hbm.at[pl.ds(base, last)])
            pltpu.sync_copy(mv_v.at[pl.ds(0, last)],
                            mv_hbm.at[pl.ds(base, last)])
            pltpu.sync_copy(idx_v.at[pl.ds(0, last)],
                            idx_hbm.at[pl.ds(base, last)])

    return sc_call


def kernel(proposals, gt_boxes, gt_labels):
    n = proposals.shape[0]
    g = gt_boxes.shape[0]
    order = jnp.argsort(gt_boxes[:, 0]).astype(jnp.int32)
    gts = gt_boxes[order]
    gtf = jnp.concatenate([gts[:, 0], gts[:, 1], gts[:, 2], gts[:, 3]])
    gti = jnp.concatenate([order, jnp.take(gt_labels, order)])
    lab, mv, idx = _make_sc_call(n, g)(
        proposals[:, 0], proposals[:, 1], proposals[:, 2], proposals[:, 3],
        gtf, gti)
    return lab, mv, idx


# speedup vs baseline: 3.1280x; 1.0130x over previous
import functools

import jax
import jax.numpy as jnp
from jax import lax
from jax.experimental import pallas as pl
from jax.experimental.pallas import tpu as pltpu
from jax.experimental.pallas import tpu_sc as plsc

L = 16
NW = 32
FG_BG_THRESH = 0.5


def _make_sc_call(n, g):
    chunk = 640
    last = n - (NW - 1) * chunk
    assert last > 0 and last % L == 0 and chunk % L == 0
    nblk = chunk // L
    nblk_last = last // L
    mesh = plsc.VectorSubcoreMesh(core_axis_name="c", subcore_axis_name="s")

    steps = []
    s = g // 2
    while s >= 1:
        steps.append(s)
        s //= 2

    @functools.partial(
        pl.kernel,
        mesh=mesh,
        compiler_params=pltpu.CompilerParams(needs_layout_passes=False),
        out_type=[
            jax.ShapeDtypeStruct((n,), jnp.int32),
            jax.ShapeDtypeStruct((n,), jnp.float32),
            jax.ShapeDtypeStruct((n,), jnp.int32),
        ],
        scratch_types=[
            pltpu.VMEM((chunk,), jnp.float32),
            pltpu.VMEM((chunk,), jnp.float32),
            pltpu.VMEM((chunk,), jnp.float32),
            pltpu.VMEM((chunk,), jnp.float32),
            pltpu.VMEM((g,), jnp.float32),
            pltpu.VMEM((g,), jnp.float32),
            pltpu.VMEM((g,), jnp.float32),
            pltpu.VMEM((g,), jnp.float32),
            pltpu.VMEM((g + L,), jnp.float32),
            pltpu.VMEM((g + L,), jnp.float32),
            pltpu.VMEM((g + L,), jnp.float32),
            pltpu.VMEM((g + L,), jnp.float32),
            pltpu.VMEM((g,), jnp.int32),
            pltpu.VMEM((g,), jnp.int32),
            pltpu.VMEM((chunk,), jnp.int32),
            pltpu.VMEM((chunk,), jnp.float32),
            pltpu.VMEM((chunk,), jnp.int32),
        ],
    )
    def sc_call(px0_hbm, py0_hbm, px1_hbm, py1_hbm,
                rx0_hbm, ry0_hbm, rx1_hbm, ry1_hbm, ord_hbm, gtl_hbm,
                lab_hbm, mv_hbm, idx_hbm,
                px0_v, py0_v, px1_v, py1_v,
                rx0_v, ry0_v, rx1_v, ry1_v,
                gx0_v, gy0_v, gx1_v, gy1_v, orig_v, gtl_v,
                lab_v, mv_v, idx_v):
        cid = lax.axis_index("c")
        sid = lax.axis_index("s")
        wid = sid * 2 + cid
        base = wid * chunk
        is_last = wid == NW - 1

        pltpu.sync_copy(rx0_hbm, rx0_v)
        pltpu.sync_copy(ry0_hbm, ry0_v)
        pltpu.sync_copy(rx1_hbm, rx1_v)
        pltpu.sync_copy(ry1_hbm, ry1_v)
        pltpu.sync_copy(ord_hbm, orig_v)
        pltpu.sync_copy(gtl_hbm, gtl_v)

        @pl.when(jnp.logical_not(is_last))
        def _():
            for hbm, v in ((px0_hbm, px0_v), (py0_hbm, py0_v),
                           (px1_hbm, px1_v), (py1_hbm, py1_v)):
                pltpu.sync_copy(hbm.at[pl.ds(base, chunk)],
                                v.at[pl.ds(0, chunk)])

        @pl.when(is_last)
        def _():
            for hbm, v in ((px0_hbm, px0_v), (py0_hbm, py0_v),
                           (px1_hbm, px1_v), (py1_hbm, py1_v)):
                pltpu.sync_copy(hbm.at[pl.ds(base, last)],
                                v.at[pl.ds(0, last)])

        big = jnp.full((L,), 1.0e30, jnp.float32)
        gx0_v[pl.ds(g, L)] = big
        gy0_v[pl.ds(g, L)] = big
        gx1_v[pl.ds(g, L)] = big
        gy1_v[pl.ds(g, L)] = big

        mw = jnp.zeros((L,), jnp.float32)
        for i in range(g // L):
            ordv = orig_v[pl.ds(i * L, L)]
            x0v = plsc.load_gather(rx0_v, [ordv])
            x1v = plsc.load_gather(rx1_v, [ordv])
            gx0_v[pl.ds(i * L, L)] = x0v
            gy0_v[pl.ds(i * L, L)] = plsc.load_gather(ry0_v, [ordv])
            gx1_v[pl.ds(i * L, L)] = x1v
            gy1_v[pl.ds(i * L, L)] = plsc.load_gather(ry1_v, [ordv])
            mw = jnp.maximum(mw, x1v - x0v)
        maxw = jnp.max(mw)

        zi = jnp.zeros((L,), jnp.int32)
        zf = jnp.zeros((L,), jnp.float32)
        c0 = zi

        def block(j, _):
            off = j * L
            px0 = px0_v[pl.ds(off, L)]
            py0 = py0_v[pl.ds(off, L)]
            px1 = px1_v[pl.ds(off, L)]
            py1 = py1_v[pl.ds(off, L)]
            parea = (px1 - px0) * (py1 - py0)

            tlo = px0 - maxw
            lo = zi
            hi = zi
            for s in steps:
                vlo = plsc.load_gather(gx0_v, [lo + (s - 1)])
                lo = jnp.where(vlo < tlo, lo + s, lo)
                vhi = plsc.load_gather(gx0_v, [hi + (s - 1)])
                hi = jnp.where(vhi <= px1, hi + s, hi)
            vlo = plsc.load_gather(gx0_v, [lo])
            lo = jnp.where(vlo < tlo, lo + 1, lo)
            vhi = plsc.load_gather(gx0_v, [hi])
            hi = jnp.where(vhi <= px1, hi + 1, hi)

            trip = jnp.max(hi - lo)
            send = jnp.full((L,), g, jnp.int32)

            def citer(k, carry):
                best, bidx = carry
                safe = jnp.minimum(lo + k, send)
                gx0 = plsc.load_gather(gx0_v, [safe])
                gy0 = plsc.load_gather(gy0_v, [safe])
                gx1 = plsc.load_gather(gx1_v, [safe])
                gy1 = plsc.load_gather(gy1_v, [safe])
                ga = (gx1 - gx0) * (gy1 - gy0)
                w = jnp.maximum(
                    jnp.minimum(px1, gx1) - jnp.maximum(px0, gx0), 0.0)
                h = jnp.maximum(
                    jnp.minimum(py1, gy1) - jnp.maximum(py0, gy0), 0.0)
                inter = w * h
                union = (parea + ga) - inter
                iou = inter / union
                upd = iou > best
                best = jnp.where(upd, iou, best)
                bidx = jnp.where(upd, safe, bidx)
                return best, bidx

            best, bidx = plsc.parallel_loop(
                0, trip, 1, unroll=4, carry=(zf, zi))(citer)

            below = best < FG_BG_THRESH
            orig = plsc.load_gather(orig_v, [bidx])
            labs = plsc.load_gather(gtl_v, [orig])
            cidx = jnp.where(below, zi, orig)
            labs = jnp.where(below, zi, labs)
            mv_v[pl.ds(off, L)] = best
            idx_v[pl.ds(off, L)] = cidx
            lab_v[pl.ds(off, L)] = labs
            return 0

        lax.fori_loop(0, jnp.where(is_last, nblk_last, nblk), block, 0)

        @pl.when(jnp.logical_not(is_last))
        def _():
            pltpu.sync_copy(lab_v.at[pl.ds(0, chunk)],
                            lab_hbm.at[pl.ds(base, chunk)])
            pltpu.sync_copy(mv_v.at[pl.ds(0, chunk)],
                            mv_hbm.at[pl.ds(base, chunk)])
            pltpu.sync_copy(idx_v.at[pl.ds(0, chunk)],
                            idx_hbm.at[pl.ds(base, chunk)])

        @pl.when(is_last)
        def _():
            pltpu.sync_copy(lab_v.at[pl.ds(0, last)],
                            lab_hbm.at[pl.ds(base, last)])
            pltpu.sync_copy(mv_v.at[pl.ds(0, last)],
                            mv_hbm.at[pl.ds(base, last)])
            pltpu.sync_copy(idx_v.at[pl.ds(0, last)],
                            idx_hbm.at[pl.ds(base, last)])

    return sc_call


def kernel(proposals, gt_boxes, gt_labels):
    n = proposals.shape[0]
    g = gt_boxes.shape[0]
    order = jnp.argsort(gt_boxes[:, 0]).astype(jnp.int32)
    lab, mv, idx = _make_sc_call(n, g)(
        proposals[:, 0], proposals[:, 1], proposals[:, 2], proposals[:, 3],
        gt_boxes[:, 0], gt_boxes[:, 1], gt_boxes[:, 2], gt_boxes[:, 3],
        order, gt_labels)
    return lab, mv, idx
